# baseline (speedup 1.0000x reference)
import functools

import jax
import jax.numpy as jnp
from jax.experimental import pallas as pl
from jax.experimental.pallas import tpu as pltpu

M = 256
BP = 800
BE = 640


def _node_body(motif_ref, xma_ref, esum_ref, wx_ref, we_ref, b_ref,
               alpha_ref, hpool_ref):
    i = pl.program_id(0)
    xma = xma_ref[...]
    logits = (jnp.dot(xma, wx_ref[...], preferred_element_type=jnp.float32)
              + jnp.dot(esum_ref[...], we_ref[...],
                        preferred_element_type=jnp.float32)
              + b_ref[0, 0])
    alpha = jax.nn.sigmoid(logits)
    weighted = alpha * xma
    em = motif_ref[0, 0, :]
    ohT = (em[None, :] == jax.lax.broadcasted_iota(jnp.int32, (M, BP), 0)
           ).astype(jnp.float32)
    partial = jnp.dot(ohT, weighted, preferred_element_type=jnp.float32)

    @pl.when(i == 0)
    def _():
        hpool_ref[...] = partial

    @pl.when(i > 0)
    def _():
        hpool_ref[...] += partial

    alpha_ref[...] = alpha


def _node_call(motif3, xma, esum, wx, we, b):
    n_blocks = motif3.shape[0]
    return pl.pallas_call(
        _node_body,
        grid=(n_blocks,),
        in_specs=[
            pl.BlockSpec((1, 1, BP), lambda i: (i, 0, 0)),
            pl.BlockSpec((BP, 256), lambda i: (i, 0)),
            pl.BlockSpec((BP, 16), lambda i: (i, 0)),
            pl.BlockSpec((256, 1), lambda i: (0, 0)),
            pl.BlockSpec((16, 1), lambda i: (0, 0)),
            pl.BlockSpec((1, 1), lambda i: (0, 0)),
        ],
        out_specs=[
            pl.BlockSpec((BP, 1), lambda i: (i, 0)),
            pl.BlockSpec((M, 256), lambda i: (0, 0)),
        ],
        out_shape=[
            jax.ShapeDtypeStruct((n_blocks * BP, 1), jnp.float32),
            jax.ShapeDtypeStruct((M, 256), jnp.float32),
        ],
    )(motif3, xma, esum, wx, we, b)


def _edge_body(em_ref, hu_ref, hv_ref, ea_ref, w1t_ref, b1_ref, w2t_ref,
               b2_ref, pat_ref, pab_ref, hpool_ref, beta_ref,
               palpha_ref, hg_ref, acc_ref, cnt_ref):
    i = pl.program_id(0)
    pi_ = jnp.concatenate([hu_ref[...], hv_ref[...], ea_ref[...]], axis=1)
    logits = (jnp.dot(pi_, pat_ref[...], preferred_element_type=jnp.float32)
              + pab_ref[0, 0])
    alpha = jax.nn.sigmoid(logits)
    hid = jnp.maximum(
        jnp.dot(pi_, w1t_ref[...], preferred_element_type=jnp.float32)
        + b1_ref[...], 0.0)
    po = (jnp.dot(hid, w2t_ref[...], preferred_element_type=jnp.float32)
          + b2_ref[...]) * alpha
    em = em_ref[0, 0, :]
    ohT = (em[None, :] == jax.lax.broadcasted_iota(jnp.int32, (M, BE), 0)
           ).astype(jnp.float32)
    partial = jnp.dot(ohT, po, preferred_element_type=jnp.float32)
    pcnt = jnp.sum(ohT, axis=1, keepdims=True)

    @pl.when(i == 0)
    def _():
        acc_ref[...] = partial
        cnt_ref[...] = pcnt

    @pl.when(i > 0)
    def _():
        acc_ref[...] += partial
        cnt_ref[...] += pcnt

    palpha_ref[...] = alpha

    @pl.when(i == pl.num_programs(0) - 1)
    def _():
        cnt = jnp.maximum(cnt_ref[...], 1.0)
        hg_ref[...] = hpool_ref[...] + beta_ref[0, 0] * (acc_ref[...] / cnt)


def _edge_call(em3, h_u, h_v, ea, w1t, b1, w2t, b2, pat, pab, hpool, beta):
    n_blocks = em3.shape[0]
    return pl.pallas_call(
        _edge_body,
        grid=(n_blocks,),
        in_specs=[
            pl.BlockSpec((1, 1, BE), lambda i: (i, 0, 0)),
            pl.BlockSpec((BE, 256), lambda i: (i, 0)),
            pl.BlockSpec((BE, 256), lambda i: (i, 0)),
            pl.BlockSpec((BE, 16), lambda i: (i, 0)),
            pl.BlockSpec((528, 512), lambda i: (0, 0)),
            pl.BlockSpec((1, 512), lambda i: (0, 0)),
            pl.BlockSpec((512, 256), lambda i: (0, 0)),
            pl.BlockSpec((1, 256), lambda i: (0, 0)),
            pl.BlockSpec((528, 1), lambda i: (0, 0)),
            pl.BlockSpec((1, 1), lambda i: (0, 0)),
            pl.BlockSpec((M, 256), lambda i: (0, 0)),
            pl.BlockSpec((1, 1), lambda i: (0, 0)),
        ],
        out_specs=[
            pl.BlockSpec((BE, 1), lambda i: (i, 0)),
            pl.BlockSpec((M, 256), lambda i: (0, 0)),
        ],
        out_shape=[
            jax.ShapeDtypeStruct((n_blocks * BE, 1), jnp.float32),
            jax.ShapeDtypeStruct((M, 256), jnp.float32),
        ],
        scratch_shapes=[
            pltpu.VMEM((M, 256), jnp.float32),
            pltpu.VMEM((M, 1), jnp.float32),
        ],
    )(em3, h_u, h_v, ea, w1t, b1, w2t, b2, pat, pab, hpool, beta)


def kernel(x, edge_index, edge_attr, motif_atom_edges, node_attn_W,
           node_attn_b, pair_attn_W, pair_attn_b, mlp_W1, mlp_b1, mlp_W2,
           mlp_b2, beta):
    P = motif_atom_edges.shape[1]
    E = edge_index.shape[1]
    motif_idx = motif_atom_edges[0]
    atom_idx = motif_atom_edges[1]

    codes = motif_idx * M + atom_idx
    cnt = jnp.zeros((M * M,), jnp.int32).at[codes].add(1)
    present = cnt > 0
    rank = jnp.cumsum(present.astype(jnp.int32)) - 1
    mapping_val = jnp.where(present, rank, -1)
    inverse = mapping_val[codes]

    u = edge_index[0]
    v = edge_index[1]
    small = (u < M) & (v < M)
    uc = jnp.where(small, u, 0)
    vc = jnp.where(small, v, 0)
    memb = present.reshape(M, M)
    common = memb[:, uc] & memb[:, vc] & small[None, :]
    has_common = common.any(axis=0)
    fcm = jnp.argmax(common.astype(jnp.float32), axis=0)
    row = jnp.where(has_common, mapping_val[fcm * M + uc], u)
    col = jnp.where(has_common, mapping_val[fcm * M + vc], v)
    edge_motif = motif_idx[inverse[row]]

    xma = x[atom_idx]
    e_sum = jnp.zeros((P, 16), jnp.float32).at[col].add(edge_attr)
    h_u = xma[row]
    h_v = xma[col]

    wx = node_attn_W[:, :256].T
    we = node_attn_W[:, 256:].T
    nb = node_attn_b.reshape(1, 1)
    motif3 = motif_idx.reshape(P // BP, 1, BP)
    node_alpha, h_pool = _node_call(motif3, xma, e_sum, wx, we, nb)

    w1t = mlp_W1.T
    b1 = mlp_b1.reshape(1, 512)
    w2t = mlp_W2.T
    b2 = mlp_b2.reshape(1, 256)
    pat = pair_attn_W.T
    pab = pair_attn_b.reshape(1, 1)
    em3 = edge_motif.reshape(E // BE, 1, BE)
    beta2 = jnp.asarray(beta, jnp.float32).reshape(1, 1)
    pair_alpha, h_G = _edge_call(em3, h_u, h_v, edge_attr, w1t, b1, w2t, b2,
                                 pat, pab, h_pool, beta2)
    return node_alpha, pair_alpha, h_G


# speedup vs baseline: 1.6720x; 1.6720x over previous
import functools

import jax
import jax.numpy as jnp
from jax.experimental import pallas as pl
from jax.experimental.pallas import tpu as pltpu

M = 256
BP = 800
BE = 640


def _relabel_body(u_ref, v_ref, bits8_ref, mapval_ref, row_ref, col_ref):
    BE2 = u_ref.shape[2]
    u = u_ref[0, 0, :]
    v = v_ref[0, 0, :]
    small = (u < M) & (v < M)
    uc = jnp.where(small, u, 0)
    vc = jnp.where(small, v, 0)
    lane256 = jax.lax.broadcasted_iota(jnp.int32, (BE2, M), 1)
    eq_u = (uc[:, None] == lane256).astype(jnp.float32)
    eq_v = (vc[:, None] == lane256).astype(jnp.float32)
    bu = jnp.dot(eq_u, bits8_ref[...], preferred_element_type=jnp.float32)
    bv = jnp.dot(eq_v, bits8_ref[...], preferred_element_type=jnp.float32)
    band = bu.astype(jnp.int32) & bv.astype(jnp.int32)
    small2 = (u[:, None] < M) & (v[:, None] < M)
    band = jnp.where(small2, band, 0)
    has = jnp.max(band, axis=1) > 0
    lb = band & -band
    j = ((lb >= 2).astype(jnp.int32) + (lb >= 4) + (lb >= 8) + (lb >= 16)
         + (lb >= 32) + (lb >= 64) + (lb >= 128))
    k8 = jax.lax.broadcasted_iota(jnp.int32, (BE2, 32), 1) * 8
    cand = jnp.where(band > 0, k8 + j, jnp.int32(16384))
    fcm = jnp.min(cand, axis=1)
    eq_f = (fcm[:, None] == lane256).astype(jnp.float32)
    t = jnp.dot(eq_f, mapval_ref[...], preferred_element_type=jnp.float32)
    row_m = jnp.sum(t * eq_u, axis=1).astype(jnp.int32)
    col_m = jnp.sum(t * eq_v, axis=1).astype(jnp.int32)
    row_ref[0, 0, :] = jnp.where(has, row_m, u)
    col_ref[0, 0, :] = jnp.where(has, col_m, v)


def _relabel_call(u3, v3, bits8, mapvalf):
    n_blocks, _, BE2 = u3.shape
    return pl.pallas_call(
        _relabel_body,
        grid=(n_blocks,),
        in_specs=[
            pl.BlockSpec((1, 1, BE2), lambda i: (i, 0, 0)),
            pl.BlockSpec((1, 1, BE2), lambda i: (i, 0, 0)),
            pl.BlockSpec((M, 32), lambda i: (0, 0)),
            pl.BlockSpec((M, M), lambda i: (0, 0)),
        ],
        out_specs=[
            pl.BlockSpec((1, 1, BE2), lambda i: (i, 0, 0)),
            pl.BlockSpec((1, 1, BE2), lambda i: (i, 0, 0)),
        ],
        out_shape=[
            jax.ShapeDtypeStruct((n_blocks, 1, BE2), jnp.int32),
            jax.ShapeDtypeStruct((n_blocks, 1, BE2), jnp.int32),
        ],
    )(u3, v3, bits8, mapvalf)


def _node_body(motif_ref, xma_ref, esum_ref, wx_ref, we_ref, b_ref,
               alpha_ref, hpool_ref):
    i = pl.program_id(0)
    xma = xma_ref[...]
    logits = (jnp.dot(xma, wx_ref[...], preferred_element_type=jnp.float32)
              + jnp.dot(esum_ref[...], we_ref[...],
                        preferred_element_type=jnp.float32)
              + b_ref[0, 0])
    alpha = jax.nn.sigmoid(logits)
    weighted = alpha * xma
    em = motif_ref[0, 0, :]
    ohT = (em[None, :] == jax.lax.broadcasted_iota(jnp.int32, (M, BP), 0)
           ).astype(jnp.float32)
    partial = jnp.dot(ohT, weighted, preferred_element_type=jnp.float32)

    @pl.when(i == 0)
    def _():
        hpool_ref[...] = partial

    @pl.when(i > 0)
    def _():
        hpool_ref[...] += partial

    alpha_ref[...] = alpha


def _node_call(motif3, xma, esum, wx, we, b):
    n_blocks = motif3.shape[0]
    return pl.pallas_call(
        _node_body,
        grid=(n_blocks,),
        in_specs=[
            pl.BlockSpec((1, 1, BP), lambda i: (i, 0, 0)),
            pl.BlockSpec((BP, 256), lambda i: (i, 0)),
            pl.BlockSpec((BP, 16), lambda i: (i, 0)),
            pl.BlockSpec((256, 1), lambda i: (0, 0)),
            pl.BlockSpec((16, 1), lambda i: (0, 0)),
            pl.BlockSpec((1, 1), lambda i: (0, 0)),
        ],
        out_specs=[
            pl.BlockSpec((BP, 1), lambda i: (i, 0)),
            pl.BlockSpec((M, 256), lambda i: (0, 0)),
        ],
        out_shape=[
            jax.ShapeDtypeStruct((n_blocks * BP, 1), jnp.float32),
            jax.ShapeDtypeStruct((M, 256), jnp.float32),
        ],
    )(motif3, xma, esum, wx, we, b)


def _edge_body(em_ref, hu_ref, hv_ref, ea_ref, w1t_ref, b1_ref, w2t_ref,
               b2_ref, pat_ref, pab_ref, hpool_ref, beta_ref,
               palpha_ref, hg_ref, acc_ref, cnt_ref):
    i = pl.program_id(0)
    pi_ = jnp.concatenate([hu_ref[...], hv_ref[...], ea_ref[...]], axis=1)
    logits = (jnp.dot(pi_, pat_ref[...], preferred_element_type=jnp.float32)
              + pab_ref[0, 0])
    alpha = jax.nn.sigmoid(logits)
    hid = jnp.maximum(
        jnp.dot(pi_, w1t_ref[...], preferred_element_type=jnp.float32)
        + b1_ref[...], 0.0)
    po = (jnp.dot(hid, w2t_ref[...], preferred_element_type=jnp.float32)
          + b2_ref[...]) * alpha
    em = em_ref[0, 0, :]
    ohT = (em[None, :] == jax.lax.broadcasted_iota(jnp.int32, (M, BE), 0)
           ).astype(jnp.float32)
    partial = jnp.dot(ohT, po, preferred_element_type=jnp.float32)
    pcnt = jnp.sum(ohT, axis=1, keepdims=True)

    @pl.when(i == 0)
    def _():
        acc_ref[...] = partial
        cnt_ref[...] = pcnt

    @pl.when(i > 0)
    def _():
        acc_ref[...] += partial
        cnt_ref[...] += pcnt

    palpha_ref[...] = alpha

    @pl.when(i == pl.num_programs(0) - 1)
    def _():
        cnt = jnp.maximum(cnt_ref[...], 1.0)
        hg_ref[...] = hpool_ref[...] + beta_ref[0, 0] * (acc_ref[...] / cnt)


def _edge_call(em3, h_u, h_v, ea, w1t, b1, w2t, b2, pat, pab, hpool, beta):
    n_blocks = em3.shape[0]
    return pl.pallas_call(
        _edge_body,
        grid=(n_blocks,),
        in_specs=[
            pl.BlockSpec((1, 1, BE), lambda i: (i, 0, 0)),
            pl.BlockSpec((BE, 256), lambda i: (i, 0)),
            pl.BlockSpec((BE, 256), lambda i: (i, 0)),
            pl.BlockSpec((BE, 16), lambda i: (i, 0)),
            pl.BlockSpec((528, 512), lambda i: (0, 0)),
            pl.BlockSpec((1, 512), lambda i: (0, 0)),
            pl.BlockSpec((512, 256), lambda i: (0, 0)),
            pl.BlockSpec((1, 256), lambda i: (0, 0)),
            pl.BlockSpec((528, 1), lambda i: (0, 0)),
            pl.BlockSpec((1, 1), lambda i: (0, 0)),
            pl.BlockSpec((M, 256), lambda i: (0, 0)),
            pl.BlockSpec((1, 1), lambda i: (0, 0)),
        ],
        out_specs=[
            pl.BlockSpec((BE, 1), lambda i: (i, 0)),
            pl.BlockSpec((M, 256), lambda i: (0, 0)),
        ],
        out_shape=[
            jax.ShapeDtypeStruct((n_blocks * BE, 1), jnp.float32),
            jax.ShapeDtypeStruct((M, 256), jnp.float32),
        ],
        scratch_shapes=[
            pltpu.VMEM((M, 256), jnp.float32),
            pltpu.VMEM((M, 1), jnp.float32),
        ],
    )(em3, h_u, h_v, ea, w1t, b1, w2t, b2, pat, pab, hpool, beta)


def kernel(x, edge_index, edge_attr, motif_atom_edges, node_attn_W,
           node_attn_b, pair_attn_W, pair_attn_b, mlp_W1, mlp_b1, mlp_W2,
           mlp_b2, beta):
    P = motif_atom_edges.shape[1]
    E = edge_index.shape[1]
    motif_idx = motif_atom_edges[0]
    atom_idx = motif_atom_edges[1]

    codes = motif_idx * M + atom_idx
    cnt = jnp.zeros((M * M,), jnp.int32).at[codes].add(1)
    present = cnt > 0
    rank = jnp.cumsum(present.astype(jnp.int32)) - 1
    mapping_val = jnp.where(present, rank, -1)
    inverse = mapping_val[codes]

    pres2 = present.reshape(M, M).astype(jnp.int32)
    bits8 = (pres2.reshape(32, 8, M)
             * (jnp.int32(1) << jnp.arange(8, dtype=jnp.int32))[None, :, None]
             ).sum(axis=1).T.astype(jnp.float32)
    mapvalf = mapping_val.reshape(M, M).astype(jnp.float32)

    u3 = edge_index[0].reshape(E // BE, 1, BE)
    v3 = edge_index[1].reshape(E // BE, 1, BE)
    row3, col3 = _relabel_call(u3, v3, bits8, mapvalf)
    row = row3.reshape(E)
    col = col3.reshape(E)
    edge_motif = motif_idx[inverse[row]]

    xma = x[atom_idx]
    e_sum = jnp.zeros((P, 16), jnp.float32).at[col].add(edge_attr)
    h_u = xma[row]
    h_v = xma[col]

    wx = node_attn_W[:, :256].T
    we = node_attn_W[:, 256:].T
    nb = node_attn_b.reshape(1, 1)
    motif3 = motif_idx.reshape(P // BP, 1, BP)
    node_alpha, h_pool = _node_call(motif3, xma, e_sum, wx, we, nb)

    w1t = mlp_W1.T
    b1 = mlp_b1.reshape(1, 512)
    w2t = mlp_W2.T
    b2 = mlp_b2.reshape(1, 256)
    pat = pair_attn_W.T
    pab = pair_attn_b.reshape(1, 1)
    em3 = edge_motif.reshape(E // BE, 1, BE)
    beta2 = jnp.asarray(beta, jnp.float32).reshape(1, 1)
    pair_alpha, h_G = _edge_call(em3, h_u, h_v, edge_attr, w1t, b1, w2t, b2,
                                 pat, pab, h_pool, beta2)
    return node_alpha, pair_alpha, h_G


# speedup vs baseline: 2.1126x; 1.2635x over previous
import functools

import jax
import jax.numpy as jnp
from jax import lax
from jax.experimental import pallas as pl
from jax.experimental.pallas import tpu as pltpu
from jax.experimental.pallas import tpu_sc as plsc

M = 256
BP = 800
BE = 640


_NC = 2
_NS = 16
_NW = _NC * _NS


def _gather2_sc(xma, row, col):
    E_, D = row.shape[0], xma.shape[1]
    rows_per_w = E_ // _NW
    ch = 200
    nch = rows_per_w // ch

    @functools.partial(
        pl.kernel,
        mesh=plsc.VectorSubcoreMesh(core_axis_name="c", subcore_axis_name="s"),
        out_type=[
            jax.ShapeDtypeStruct((E_, D), jnp.float32),
            jax.ShapeDtypeStruct((E_, D), jnp.float32),
        ],
        scratch_types=[
            pltpu.VMEM((rows_per_w,), jnp.int32),
            pltpu.VMEM((rows_per_w,), jnp.int32),
            pltpu.VMEM((ch, D), jnp.float32),
            pltpu.VMEM((ch, D), jnp.float32),
            pltpu.SemaphoreType.DMA,
            pltpu.SemaphoreType.DMA,
        ],
    )
    def k(xma_hbm, rowi_hbm, coli_hbm, hu_hbm, hv_hbm,
          idxu_v, idxv_v, bufu, bufv, semu, semv):
        wid = lax.axis_index("s") * _NC + lax.axis_index("c")
        base = wid * rows_per_w
        pltpu.sync_copy(rowi_hbm.at[pl.ds(base, rows_per_w)], idxu_v)
        pltpu.sync_copy(coli_hbm.at[pl.ds(base, rows_per_w)], idxv_v)

        def body(c, carry):
            off = pl.multiple_of(c * ch, 8)
            cpu = pltpu.make_async_copy(
                xma_hbm.at[idxu_v.at[pl.ds(off, ch)]], bufu, semu)
            cpv = pltpu.make_async_copy(
                xma_hbm.at[idxv_v.at[pl.ds(off, ch)]], bufv, semv)
            cpu.start()
            cpv.start()
            cpu.wait()
            pltpu.sync_copy(bufu, hu_hbm.at[pl.ds(base + off, ch)])
            cpv.wait()
            pltpu.sync_copy(bufv, hv_hbm.at[pl.ds(base + off, ch)])
            return carry

        lax.fori_loop(0, nch, body, 0)

    return k(xma, row, col)


def _relabel_body(u_ref, v_ref, bits8_ref, mapval_ref, row_ref, col_ref):
    BE2 = u_ref.shape[2]
    u = u_ref[0, 0, :]
    v = v_ref[0, 0, :]
    small = (u < M) & (v < M)
    uc = jnp.where(small, u, 0)
    vc = jnp.where(small, v, 0)
    lane256 = jax.lax.broadcasted_iota(jnp.int32, (BE2, M), 1)
    eq_u = (uc[:, None] == lane256).astype(jnp.float32)
    eq_v = (vc[:, None] == lane256).astype(jnp.float32)
    bu = jnp.dot(eq_u, bits8_ref[...], preferred_element_type=jnp.float32)
    bv = jnp.dot(eq_v, bits8_ref[...], preferred_element_type=jnp.float32)
    band = bu.astype(jnp.int32) & bv.astype(jnp.int32)
    small2 = (u[:, None] < M) & (v[:, None] < M)
    band = jnp.where(small2, band, 0)
    has = jnp.max(band, axis=1) > 0
    lb = band & -band
    j = ((lb >= 2).astype(jnp.int32) + (lb >= 4) + (lb >= 8) + (lb >= 16)
         + (lb >= 32) + (lb >= 64) + (lb >= 128))
    k8 = jax.lax.broadcasted_iota(jnp.int32, (BE2, 32), 1) * 8
    cand = jnp.where(band > 0, k8 + j, jnp.int32(16384))
    fcm = jnp.min(cand, axis=1)
    eq_f = (fcm[:, None] == lane256).astype(jnp.float32)
    t = jnp.dot(eq_f, mapval_ref[...], preferred_element_type=jnp.float32)
    row_m = jnp.sum(t * eq_u, axis=1).astype(jnp.int32)
    col_m = jnp.sum(t * eq_v, axis=1).astype(jnp.int32)
    row_ref[0, 0, :] = jnp.where(has, row_m, u)
    col_ref[0, 0, :] = jnp.where(has, col_m, v)


def _relabel_call(u3, v3, bits8, mapvalf):
    n_blocks, _, BE2 = u3.shape
    return pl.pallas_call(
        _relabel_body,
        grid=(n_blocks,),
        in_specs=[
            pl.BlockSpec((1, 1, BE2), lambda i: (i, 0, 0)),
            pl.BlockSpec((1, 1, BE2), lambda i: (i, 0, 0)),
            pl.BlockSpec((M, 32), lambda i: (0, 0)),
            pl.BlockSpec((M, M), lambda i: (0, 0)),
        ],
        out_specs=[
            pl.BlockSpec((1, 1, BE2), lambda i: (i, 0, 0)),
            pl.BlockSpec((1, 1, BE2), lambda i: (i, 0, 0)),
        ],
        out_shape=[
            jax.ShapeDtypeStruct((n_blocks, 1, BE2), jnp.int32),
            jax.ShapeDtypeStruct((n_blocks, 1, BE2), jnp.int32),
        ],
    )(u3, v3, bits8, mapvalf)


def _node_body(motif_ref, xma_ref, esum_ref, wx_ref, we_ref, b_ref,
               alpha_ref, hpool_ref):
    i = pl.program_id(0)
    xma = xma_ref[...]
    logits = (jnp.dot(xma, wx_ref[...], preferred_element_type=jnp.float32)
              + jnp.dot(esum_ref[...], we_ref[...],
                        preferred_element_type=jnp.float32)
              + b_ref[0, 0])
    alpha = jax.nn.sigmoid(logits)
    weighted = alpha * xma
    em = motif_ref[0, 0, :]
    ohT = (em[None, :] == jax.lax.broadcasted_iota(jnp.int32, (M, BP), 0)
           ).astype(jnp.float32)
    partial = jnp.dot(ohT, weighted, preferred_element_type=jnp.float32)

    @pl.when(i == 0)
    def _():
        hpool_ref[...] = partial

    @pl.when(i > 0)
    def _():
        hpool_ref[...] += partial

    alpha_ref[...] = alpha


def _node_call(motif3, xma, esum, wx, we, b):
    n_blocks = motif3.shape[0]
    return pl.pallas_call(
        _node_body,
        grid=(n_blocks,),
        in_specs=[
            pl.BlockSpec((1, 1, BP), lambda i: (i, 0, 0)),
            pl.BlockSpec((BP, 256), lambda i: (i, 0)),
            pl.BlockSpec((BP, 16), lambda i: (i, 0)),
            pl.BlockSpec((256, 1), lambda i: (0, 0)),
            pl.BlockSpec((16, 1), lambda i: (0, 0)),
            pl.BlockSpec((1, 1), lambda i: (0, 0)),
        ],
        out_specs=[
            pl.BlockSpec((BP, 1), lambda i: (i, 0)),
            pl.BlockSpec((M, 256), lambda i: (0, 0)),
        ],
        out_shape=[
            jax.ShapeDtypeStruct((n_blocks * BP, 1), jnp.float32),
            jax.ShapeDtypeStruct((M, 256), jnp.float32),
        ],
    )(motif3, xma, esum, wx, we, b)


def _edge_body(em_ref, hu_ref, hv_ref, ea_ref, w1t_ref, b1_ref, w2t_ref,
               b2_ref, pat_ref, pab_ref, hpool_ref, beta_ref,
               palpha_ref, hg_ref, acc_ref, cnt_ref):
    i = pl.program_id(0)
    pi_ = jnp.concatenate([hu_ref[...], hv_ref[...], ea_ref[...]], axis=1)
    logits = (jnp.dot(pi_, pat_ref[...], preferred_element_type=jnp.float32)
              + pab_ref[0, 0])
    alpha = jax.nn.sigmoid(logits)
    hid = jnp.maximum(
        jnp.dot(pi_, w1t_ref[...], preferred_element_type=jnp.float32)
        + b1_ref[...], 0.0)
    po = (jnp.dot(hid, w2t_ref[...], preferred_element_type=jnp.float32)
          + b2_ref[...]) * alpha
    em = em_ref[0, 0, :]
    ohT = (em[None, :] == jax.lax.broadcasted_iota(jnp.int32, (M, BE), 0)
           ).astype(jnp.float32)
    partial = jnp.dot(ohT, po, preferred_element_type=jnp.float32)
    pcnt = jnp.sum(ohT, axis=1, keepdims=True)

    @pl.when(i == 0)
    def _():
        acc_ref[...] = partial
        cnt_ref[...] = pcnt

    @pl.when(i > 0)
    def _():
        acc_ref[...] += partial
        cnt_ref[...] += pcnt

    palpha_ref[...] = alpha

    @pl.when(i == pl.num_programs(0) - 1)
    def _():
        cnt = jnp.maximum(cnt_ref[...], 1.0)
        hg_ref[...] = hpool_ref[...] + beta_ref[0, 0] * (acc_ref[...] / cnt)


def _edge_call(em3, h_u, h_v, ea, w1t, b1, w2t, b2, pat, pab, hpool, beta):
    n_blocks = em3.shape[0]
    return pl.pallas_call(
        _edge_body,
        grid=(n_blocks,),
        in_specs=[
            pl.BlockSpec((1, 1, BE), lambda i: (i, 0, 0)),
            pl.BlockSpec((BE, 256), lambda i: (i, 0)),
            pl.BlockSpec((BE, 256), lambda i: (i, 0)),
            pl.BlockSpec((BE, 16), lambda i: (i, 0)),
            pl.BlockSpec((528, 512), lambda i: (0, 0)),
            pl.BlockSpec((1, 512), lambda i: (0, 0)),
            pl.BlockSpec((512, 256), lambda i: (0, 0)),
            pl.BlockSpec((1, 256), lambda i: (0, 0)),
            pl.BlockSpec((528, 1), lambda i: (0, 0)),
            pl.BlockSpec((1, 1), lambda i: (0, 0)),
            pl.BlockSpec((M, 256), lambda i: (0, 0)),
            pl.BlockSpec((1, 1), lambda i: (0, 0)),
        ],
        out_specs=[
            pl.BlockSpec((BE, 1), lambda i: (i, 0)),
            pl.BlockSpec((M, 256), lambda i: (0, 0)),
        ],
        out_shape=[
            jax.ShapeDtypeStruct((n_blocks * BE, 1), jnp.float32),
            jax.ShapeDtypeStruct((M, 256), jnp.float32),
        ],
        scratch_shapes=[
            pltpu.VMEM((M, 256), jnp.float32),
            pltpu.VMEM((M, 1), jnp.float32),
        ],
    )(em3, h_u, h_v, ea, w1t, b1, w2t, b2, pat, pab, hpool, beta)


def kernel(x, edge_index, edge_attr, motif_atom_edges, node_attn_W,
           node_attn_b, pair_attn_W, pair_attn_b, mlp_W1, mlp_b1, mlp_W2,
           mlp_b2, beta):
    P = motif_atom_edges.shape[1]
    E = edge_index.shape[1]
    motif_idx = motif_atom_edges[0]
    atom_idx = motif_atom_edges[1]

    codes = motif_idx * M + atom_idx
    cnt = jnp.zeros((M * M,), jnp.int32).at[codes].add(1)
    present = cnt > 0
    rank = jnp.cumsum(present.astype(jnp.int32)) - 1
    mapping_val = jnp.where(present, rank, -1)
    inverse = mapping_val[codes]

    pres2 = present.reshape(M, M).astype(jnp.int32)
    bits8 = (pres2.reshape(32, 8, M)
             * (jnp.int32(1) << jnp.arange(8, dtype=jnp.int32))[None, :, None]
             ).sum(axis=1).T.astype(jnp.float32)
    mapvalf = mapping_val.reshape(M, M).astype(jnp.float32)

    u3 = edge_index[0].reshape(E // BE, 1, BE)
    v3 = edge_index[1].reshape(E // BE, 1, BE)
    row3, col3 = _relabel_call(u3, v3, bits8, mapvalf)
    row = row3.reshape(E)
    col = col3.reshape(E)
    edge_motif = motif_idx[inverse[row]]

    xma = x[atom_idx]
    e_sum = jnp.zeros((P, 16), jnp.float32).at[col].add(edge_attr)
    h_u, h_v = _gather2_sc(xma, row, col)

    wx = node_attn_W[:, :256].T
    we = node_attn_W[:, 256:].T
    nb = node_attn_b.reshape(1, 1)
    motif3 = motif_idx.reshape(P // BP, 1, BP)
    node_alpha, h_pool = _node_call(motif3, xma, e_sum, wx, we, nb)

    w1t = mlp_W1.T
    b1 = mlp_b1.reshape(1, 512)
    w2t = mlp_W2.T
    b2 = mlp_b2.reshape(1, 256)
    pat = pair_attn_W.T
    pab = pair_attn_b.reshape(1, 1)
    em3 = edge_motif.reshape(E // BE, 1, BE)
    beta2 = jnp.asarray(beta, jnp.float32).reshape(1, 1)
    pair_alpha, h_G = _edge_call(em3, h_u, h_v, edge_attr, w1t, b1, w2t, b2,
                                 pat, pab, h_pool, beta2)
    return node_alpha, pair_alpha, h_G


# speedup vs baseline: 2.1149x; 1.0011x over previous
import functools

import jax
import jax.numpy as jnp
from jax import lax
from jax.experimental import pallas as pl
from jax.experimental.pallas import tpu as pltpu
from jax.experimental.pallas import tpu_sc as plsc

M = 256
BP = 800
BE = 640


_NC = 2
_NS = 16
_NW = _NC * _NS


def _gather2_sc(xma, row, col):
    E_, D = row.shape[0], xma.shape[1]
    rows_per_w = E_ // _NW
    ch = 200
    nch = rows_per_w // ch

    @functools.partial(
        pl.kernel,
        mesh=plsc.VectorSubcoreMesh(core_axis_name="c", subcore_axis_name="s"),
        out_type=[
            jax.ShapeDtypeStruct((E_, D), jnp.float32),
            jax.ShapeDtypeStruct((E_, D), jnp.float32),
        ],
        scratch_types=[
            pltpu.VMEM((rows_per_w,), jnp.int32),
            pltpu.VMEM((rows_per_w,), jnp.int32),
            pltpu.VMEM((ch, D), jnp.float32),
            pltpu.VMEM((ch, D), jnp.float32),
            pltpu.SemaphoreType.DMA,
            pltpu.SemaphoreType.DMA,
        ],
    )
    def k(xma_hbm, rowi_hbm, coli_hbm, hu_hbm, hv_hbm,
          idxu_v, idxv_v, bufu, bufv, semu, semv):
        wid = lax.axis_index("s") * _NC + lax.axis_index("c")
        base = wid * rows_per_w
        pltpu.sync_copy(rowi_hbm.at[pl.ds(base, rows_per_w)], idxu_v)
        pltpu.sync_copy(coli_hbm.at[pl.ds(base, rows_per_w)], idxv_v)

        def body(c, carry):
            off = pl.multiple_of(c * ch, 8)
            cpu = pltpu.make_async_copy(
                xma_hbm.at[idxu_v.at[pl.ds(off, ch)]], bufu, semu)
            cpv = pltpu.make_async_copy(
                xma_hbm.at[idxv_v.at[pl.ds(off, ch)]], bufv, semv)
            cpu.start()
            cpv.start()
            cpu.wait()
            pltpu.sync_copy(bufu, hu_hbm.at[pl.ds(base + off, ch)])
            cpv.wait()
            pltpu.sync_copy(bufv, hv_hbm.at[pl.ds(base + off, ch)])
            return carry

        lax.fori_loop(0, nch, body, 0)

    return k(xma, row, col)


def _relabel_body(u_ref, v_ref, bits8_ref, mapval_ref, row_ref, col_ref):
    BE2 = u_ref.shape[2]
    u = u_ref[0, 0, :]
    v = v_ref[0, 0, :]
    small = (u < M) & (v < M)
    uc = jnp.where(small, u, 0)
    vc = jnp.where(small, v, 0)
    lane256 = jax.lax.broadcasted_iota(jnp.int32, (BE2, M), 1)
    eq_u = (uc[:, None] == lane256).astype(jnp.float32)
    eq_v = (vc[:, None] == lane256).astype(jnp.float32)
    bu = jnp.dot(eq_u, bits8_ref[...], preferred_element_type=jnp.float32)
    bv = jnp.dot(eq_v, bits8_ref[...], preferred_element_type=jnp.float32)
    band = bu.astype(jnp.int32) & bv.astype(jnp.int32)
    small2 = (u[:, None] < M) & (v[:, None] < M)
    band = jnp.where(small2, band, 0)
    has = jnp.max(band, axis=1) > 0
    lb = band & -band
    j = ((lb >= 2).astype(jnp.int32) + (lb >= 4) + (lb >= 8) + (lb >= 16)
         + (lb >= 32) + (lb >= 64) + (lb >= 128))
    k8 = jax.lax.broadcasted_iota(jnp.int32, (BE2, 32), 1) * 8
    cand = jnp.where(band > 0, k8 + j, jnp.int32(16384))
    fcm = jnp.min(cand, axis=1)
    eq_f = (fcm[:, None] == lane256).astype(jnp.float32)
    t = jnp.dot(eq_f, mapval_ref[...], preferred_element_type=jnp.float32)
    row_m = jnp.sum(t * eq_u, axis=1).astype(jnp.int32)
    col_m = jnp.sum(t * eq_v, axis=1).astype(jnp.int32)
    row_ref[0, 0, :] = jnp.where(has, row_m, u)
    col_ref[0, 0, :] = jnp.where(has, col_m, v)


def _relabel_call(u3, v3, bits8, mapvalf):
    n_blocks, _, BE2 = u3.shape
    return pl.pallas_call(
        _relabel_body,
        grid=(n_blocks,),
        in_specs=[
            pl.BlockSpec((1, 1, BE2), lambda i: (i, 0, 0)),
            pl.BlockSpec((1, 1, BE2), lambda i: (i, 0, 0)),
            pl.BlockSpec((M, 32), lambda i: (0, 0)),
            pl.BlockSpec((M, M), lambda i: (0, 0)),
        ],
        out_specs=[
            pl.BlockSpec((1, 1, BE2), lambda i: (i, 0, 0)),
            pl.BlockSpec((1, 1, BE2), lambda i: (i, 0, 0)),
        ],
        out_shape=[
            jax.ShapeDtypeStruct((n_blocks, 1, BE2), jnp.int32),
            jax.ShapeDtypeStruct((n_blocks, 1, BE2), jnp.int32),
        ],
    )(u3, v3, bits8, mapvalf)


def _node_body(motif_ref, xma_ref, esum_ref, wx_ref, we_ref, b_ref,
               alpha_ref, hpool_ref):
    i = pl.program_id(0)
    xma = xma_ref[...]
    logits = (jnp.dot(xma, wx_ref[...], preferred_element_type=jnp.float32)
              + jnp.dot(esum_ref[...], we_ref[...],
                        preferred_element_type=jnp.float32)
              + b_ref[0, 0])
    alpha = jax.nn.sigmoid(logits)
    weighted = alpha * xma
    em = motif_ref[0, 0, :]
    ohT = (em[None, :] == jax.lax.broadcasted_iota(jnp.int32, (M, BP), 0)
           ).astype(jnp.float32)
    partial = jnp.dot(ohT, weighted, preferred_element_type=jnp.float32)

    @pl.when(i == 0)
    def _():
        hpool_ref[...] = partial

    @pl.when(i > 0)
    def _():
        hpool_ref[...] += partial

    alpha_ref[...] = alpha


def _node_call(motif3, xma, esum, wx, we, b):
    n_blocks = motif3.shape[0]
    return pl.pallas_call(
        _node_body,
        grid=(n_blocks,),
        in_specs=[
            pl.BlockSpec((1, 1, BP), lambda i: (i, 0, 0)),
            pl.BlockSpec((BP, 256), lambda i: (i, 0)),
            pl.BlockSpec((BP, 16), lambda i: (i, 0)),
            pl.BlockSpec((256, 1), lambda i: (0, 0)),
            pl.BlockSpec((16, 1), lambda i: (0, 0)),
            pl.BlockSpec((1, 1), lambda i: (0, 0)),
        ],
        out_specs=[
            pl.BlockSpec((BP, 1), lambda i: (i, 0)),
            pl.BlockSpec((M, 256), lambda i: (0, 0)),
        ],
        out_shape=[
            jax.ShapeDtypeStruct((n_blocks * BP, 1), jnp.float32),
            jax.ShapeDtypeStruct((M, 256), jnp.float32),
        ],
    )(motif3, xma, esum, wx, we, b)


def _edge_body(em_ref, hu_ref, hv_ref, ea_ref, w1t_ref, b1_ref, w2t_ref,
               b2_ref, pat_ref, pab_ref, hpool_ref, beta_ref,
               palpha_ref, hg_ref, acc_ref, cnt_ref):
    i = pl.program_id(0)
    pi_ = jnp.concatenate([hu_ref[...], hv_ref[...], ea_ref[...]], axis=1)
    logits = (jnp.dot(pi_, pat_ref[...], preferred_element_type=jnp.float32)
              + pab_ref[0, 0])
    alpha = jax.nn.sigmoid(logits)
    hid = jnp.maximum(
        jnp.dot(pi_, w1t_ref[...], preferred_element_type=jnp.float32)
        + b1_ref[...], 0.0)
    po = (jnp.dot(hid, w2t_ref[...], preferred_element_type=jnp.float32)
          + b2_ref[...]) * alpha
    em = em_ref[0, 0, :]
    ohT = (em[None, :] == jax.lax.broadcasted_iota(jnp.int32, (M, BE), 0)
           ).astype(jnp.float32)
    partial = jnp.dot(ohT, po, preferred_element_type=jnp.float32)
    pcnt = jnp.sum(ohT, axis=1, keepdims=True)

    @pl.when(i == 0)
    def _():
        acc_ref[...] = partial
        cnt_ref[...] = pcnt

    @pl.when(i > 0)
    def _():
        acc_ref[...] += partial
        cnt_ref[...] += pcnt

    palpha_ref[...] = alpha

    @pl.when(i == pl.num_programs(0) - 1)
    def _():
        cnt = jnp.maximum(cnt_ref[...], 1.0)
        hg_ref[...] = hpool_ref[...] + beta_ref[0, 0] * (acc_ref[...] / cnt)


def _edge_call(em3, h_u, h_v, ea, w1t, b1, w2t, b2, pat, pab, hpool, beta):
    n_blocks = em3.shape[0]
    return pl.pallas_call(
        _edge_body,
        grid=(n_blocks,),
        in_specs=[
            pl.BlockSpec((1, 1, BE), lambda i: (i, 0, 0)),
            pl.BlockSpec((BE, 256), lambda i: (i, 0)),
            pl.BlockSpec((BE, 256), lambda i: (i, 0)),
            pl.BlockSpec((BE, 16), lambda i: (i, 0)),
            pl.BlockSpec((528, 512), lambda i: (0, 0)),
            pl.BlockSpec((1, 512), lambda i: (0, 0)),
            pl.BlockSpec((512, 256), lambda i: (0, 0)),
            pl.BlockSpec((1, 256), lambda i: (0, 0)),
            pl.BlockSpec((528, 1), lambda i: (0, 0)),
            pl.BlockSpec((1, 1), lambda i: (0, 0)),
            pl.BlockSpec((M, 256), lambda i: (0, 0)),
            pl.BlockSpec((1, 1), lambda i: (0, 0)),
        ],
        out_specs=[
            pl.BlockSpec((BE, 1), lambda i: (i, 0)),
            pl.BlockSpec((M, 256), lambda i: (0, 0)),
        ],
        out_shape=[
            jax.ShapeDtypeStruct((n_blocks * BE, 1), jnp.float32),
            jax.ShapeDtypeStruct((M, 256), jnp.float32),
        ],
        scratch_shapes=[
            pltpu.VMEM((M, 256), jnp.float32),
            pltpu.VMEM((M, 1), jnp.float32),
        ],
    )(em3, h_u, h_v, ea, w1t, b1, w2t, b2, pat, pab, hpool, beta)


def kernel(x, edge_index, edge_attr, motif_atom_edges, node_attn_W,
           node_attn_b, pair_attn_W, pair_attn_b, mlp_W1, mlp_b1, mlp_W2,
           mlp_b2, beta):
    P = motif_atom_edges.shape[1]
    E = edge_index.shape[1]
    motif_idx = motif_atom_edges[0]
    atom_idx = motif_atom_edges[1]

    codes = motif_idx * M + atom_idx
    cnt = jnp.zeros((M * M,), jnp.int32).at[codes].add(1)
    present = cnt > 0
    rank = jnp.cumsum(present.astype(jnp.int32)) - 1
    mapping_val = jnp.where(present, rank, -1)
    inverse = mapping_val[codes]

    pres2 = present.reshape(M, M).astype(jnp.int32)
    bits8 = (pres2.reshape(32, 8, M)
             * (jnp.int32(1) << jnp.arange(8, dtype=jnp.int32))[None, :, None]
             ).sum(axis=1).T.astype(jnp.float32)
    mapvalf = mapping_val.reshape(M, M).astype(jnp.float32)

    u3 = edge_index[0].reshape(E // BE, 1, BE)
    v3 = edge_index[1].reshape(E // BE, 1, BE)
    row3, col3 = _relabel_call(u3, v3, bits8, mapvalf)
    row = row3.reshape(E)
    col = col3.reshape(E)
    edge_motif = motif_idx[inverse[row]]

    xma = x[atom_idx]
    h_u, h_v = _gather2_sc(xma, row, col)
    e_sum = jnp.zeros((P, 16), jnp.float32).at[col].add(edge_attr)

    wx = node_attn_W[:, :256].T
    we = node_attn_W[:, 256:].T
    nb = node_attn_b.reshape(1, 1)
    motif3 = motif_idx.reshape(P // BP, 1, BP)
    node_alpha, h_pool = _node_call(motif3, xma, e_sum, wx, we, nb)

    w1t = mlp_W1.T
    b1 = mlp_b1.reshape(1, 512)
    w2t = mlp_W2.T
    b2 = mlp_b2.reshape(1, 256)
    pat = pair_attn_W.T
    pab = pair_attn_b.reshape(1, 1)
    em3 = edge_motif.reshape(E // BE, 1, BE)
    beta2 = jnp.asarray(beta, jnp.float32).reshape(1, 1)
    pair_alpha, h_G = _edge_call(em3, h_u, h_v, edge_attr, w1t, b1, w2t, b2,
                                 pat, pab, h_pool, beta2)
    return node_alpha, pair_alpha, h_G
